# speedup vs baseline: 1.6410x; 1.6410x over previous
import functools

import jax
import jax.numpy as jnp
from jax import lax
from jax.experimental import pallas as pl
from jax.experimental.pallas import tpu as pltpu
from jax.experimental.pallas import tpu_sc as plsc

_NC = 2
_NS = 16
_NW = _NC * _NS

_GATHER_CHUNK = 64


def _sc_gather(pvs, idx):
    b, d = idx.shape[0], pvs.shape[1]
    b_per_w = b // _NW
    n_chunks = b_per_w // _GATHER_CHUNK

    @functools.partial(
        pl.kernel,
        mesh=plsc.VectorSubcoreMesh(core_axis_name="c", subcore_axis_name="s"),
        out_type=jax.ShapeDtypeStruct((b, d), jnp.float32),
        scratch_types=[
            pltpu.VMEM((b_per_w,), jnp.int32),
            pltpu.VMEM((_GATHER_CHUNK, d), jnp.float32),
            pltpu.SemaphoreType.DMA,
        ],
    )
    def gather_kernel(table_hbm, idx_hbm, out_hbm, idx_v, rows_v, sem):
        wid = lax.axis_index("s") * _NC + lax.axis_index("c")
        base = wid * b_per_w
        pltpu.sync_copy(idx_hbm.at[pl.ds(base, b_per_w)], idx_v)

        @pl.loop(0, n_chunks)
        def _(ci):
            off = ci * _GATHER_CHUNK
            pltpu.async_copy(
                table_hbm.at[idx_v.at[pl.ds(off, _GATHER_CHUNK)]], rows_v, sem
            ).wait()
            pltpu.sync_copy(rows_v, out_hbm.at[pl.ds(base + off, _GATHER_CHUNK)])

    return gather_kernel(pvs, idx)


def _compute_body(qv_ref, qc_ref, pv_ref, qw_ref, lab_ref, out_ref):
    qw = qw_ref[...]
    bundled = (
        qv_ref[...] * qw[0:1, :]
        + qc_ref[...] * qw[1:2, :]
        + pv_ref[...] * qw[2:3, :]
    )
    q = jnp.tanh(bundled)
    inv_nq = 1.0 / (jnp.sqrt(jnp.sum(q * q, axis=1, keepdims=True)) + 1e-12)
    lab = lab_ref[...]
    inv_nl = 1.0 / (jnp.sqrt(jnp.sum(lab * lab, axis=1, keepdims=True)) + 1e-12)
    cols = [
        jnp.sum(q * lab[l : l + 1, :], axis=1, keepdims=True) * inv_nl[l, 0]
        for l in range(3)
    ]
    out_ref[...] = jnp.concatenate(cols, axis=1) * inv_nq


def _tc_compute(qv, qc, pv, qw, label):
    batch, d = qv.shape
    bb = 512
    return pl.pallas_call(
        _compute_body,
        grid=(batch // bb,),
        in_specs=[
            pl.BlockSpec((bb, d), lambda i: (i, 0)),
            pl.BlockSpec((bb, d), lambda i: (i, 0)),
            pl.BlockSpec((bb, d), lambda i: (i, 0)),
            pl.BlockSpec((3, d), lambda i: (0, 0)),
            pl.BlockSpec((3, d), lambda i: (0, 0)),
        ],
        out_specs=pl.BlockSpec((bb, 3), lambda i: (i, 0)),
        out_shape=jax.ShapeDtypeStruct((batch, 3), jnp.float32),
    )(qv, qc, pv, qw, label)


def kernel(query_vec, qclass_vec, pvs, query_weight, label, product_idx):
    pv = _sc_gather(pvs, product_idx.astype(jnp.int32))
    return _tc_compute(query_vec, qclass_vec, pv, query_weight, label)
